# baseline (speedup 1.0000x reference)
import functools

import jax
import jax.numpy as jnp
from jax import lax
from jax.experimental import pallas as pl
from jax.experimental.pallas import tpu as pltpu
from jax.experimental.pallas import tpu_sc as plsc

_VOCAB = 1000000
_D = 64
_SEQ = 200
_BATCH = 1024
_L = 16

_NC = 2
_NS = 16
_NW = _NC * _NS
_SEQ_PER_W = _BATCH // _NW


def _build_sc_call():
    mesh = plsc.VectorSubcoreMesh(core_axis_name="c", subcore_axis_name="s")

    @functools.partial(
        pl.kernel,
        mesh=mesh,
        compiler_params=pltpu.CompilerParams(use_tc_tiling_on_sc=False),
        out_type=jax.ShapeDtypeStruct((_BATCH * _SEQ, _D), jnp.float32),
        scratch_types=[
            pltpu.VMEM((_SEQ,), jnp.int32),
            pltpu.VMEM((_SEQ,), jnp.int32),
            pltpu.VMEM((_SEQ, _D), jnp.float32),
            pltpu.VMEM((_SEQ, _D), jnp.float32),
            pltpu.SemaphoreType.DMA,
        ],
    )
    def sc_gather(ids_hbm, remap_hbm, emb_hbm, wpe_hbm, out_hbm,
                  ids_v, dict_v, rows_v, wpe_v, sem):
        wid = lax.axis_index("s") * _NC + lax.axis_index("c")
        seq0 = wid * _SEQ_PER_W

        pltpu.sync_copy(wpe_hbm, wpe_v)

        def per_seq(i, carry):
            seq = seq0 + i
            base = seq * _SEQ
            pltpu.sync_copy(ids_hbm.at[pl.ds(base, _SEQ)], ids_v)
            pltpu.async_copy(remap_hbm.at[ids_v], dict_v, sem).wait()
            pltpu.async_copy(emb_hbm.at[dict_v], rows_v, sem).wait()

            def add_pos(p, c2):
                for d in range(_D // _L):
                    sl = pl.ds(d * _L, _L)
                    rows_v[p, sl] = rows_v[p, sl] + wpe_v[p, sl]
                return c2

            lax.fori_loop(0, _SEQ, add_pos, 0)
            pltpu.sync_copy(rows_v, out_hbm.at[pl.ds(base, _SEQ)])
            return carry

        lax.fori_loop(0, _SEQ_PER_W, per_seq, 0)

    return sc_gather


_SC_CALL = _build_sc_call()


def kernel(input_ids, attention_mask, embedding_dict, input_ids2dict_ids, wpe):
    ids_flat = input_ids.reshape(_BATCH * _SEQ)
    out_flat = _SC_CALL(ids_flat, input_ids2dict_ids, embedding_dict, wpe)
    return out_flat.reshape(_BATCH, _SEQ, _D), attention_mask


# speedup vs baseline: 1.0940x; 1.0940x over previous
import functools

import jax
import jax.numpy as jnp
from jax import lax
from jax.experimental import pallas as pl
from jax.experimental.pallas import tpu as pltpu
from jax.experimental.pallas import tpu_sc as plsc

_VOCAB = 1000000
_D = 64
_SEQ = 200
_BATCH = 1024
_L = 16

_NC = 2
_NS = 16
_NW = _NC * _NS
_SEQ_PER_W = _BATCH // _NW
_CSEQ = 2
_CROWS = _CSEQ * _SEQ
_NCHUNK = _SEQ_PER_W // _CSEQ


def _build_sc_call():
    mesh = plsc.VectorSubcoreMesh(core_axis_name="c", subcore_axis_name="s")

    @functools.partial(
        pl.kernel,
        mesh=mesh,
        compiler_params=pltpu.CompilerParams(use_tc_tiling_on_sc=False),
        out_type=jax.ShapeDtypeStruct((_BATCH * _SEQ, _D), jnp.float32),
        scratch_types=[
            pltpu.VMEM((2, _CROWS), jnp.int32),
            pltpu.VMEM((2, _CROWS), jnp.int32),
            pltpu.VMEM((2, _CROWS, _D), jnp.float32),
            pltpu.VMEM((_SEQ, _D), jnp.float32),
            pltpu.SemaphoreType.DMA,
            pltpu.SemaphoreType.DMA,
            pltpu.SemaphoreType.DMA,
            pltpu.SemaphoreType.DMA,
            pltpu.SemaphoreType.DMA,
            pltpu.SemaphoreType.DMA,
        ],
    )
    def sc_gather(ids_hbm, remap_hbm, emb_hbm, wpe_hbm, out_hbm,
                  ids_v, dict_v, rows_v, wpe_v,
                  sem_r0, sem_r1, sem_e0, sem_e1, sem_o0, sem_o1):
        wid = lax.axis_index("s") * _NC + lax.axis_index("c")
        chunk0 = wid * _NCHUNK
        sem_r = (sem_r0, sem_r1)
        sem_e = (sem_e0, sem_e1)
        sem_o = (sem_o0, sem_o1)

        pltpu.sync_copy(wpe_hbm, wpe_v)

        def start_remap(i, b):
            base = (chunk0 + i) * _CROWS
            pltpu.sync_copy(ids_hbm.at[pl.ds(base, _CROWS)], ids_v.at[b])
            pltpu.make_async_copy(
                remap_hbm.at[ids_v.at[b]], dict_v.at[b], sem_r[b]).start()

        def wait_remap(b):
            pltpu.make_async_copy(
                remap_hbm.at[ids_v.at[b]], dict_v.at[b], sem_r[b]).wait()

        def start_emb(b):
            pltpu.make_async_copy(
                emb_hbm.at[dict_v.at[b]], rows_v.at[b], sem_e[b]).start()

        def wait_emb(b):
            pltpu.make_async_copy(
                emb_hbm.at[dict_v.at[b]], rows_v.at[b], sem_e[b]).wait()

        def start_out(i, b):
            base = (chunk0 + i) * _CROWS
            pltpu.make_async_copy(
                rows_v.at[b], out_hbm.at[pl.ds(base, _CROWS)], sem_o[b]).start()

        def wait_out(i, b):
            base = (chunk0 + i) * _CROWS
            pltpu.make_async_copy(
                rows_v.at[b], out_hbm.at[pl.ds(base, _CROWS)], sem_o[b]).wait()

        def add_pos(b):
            def body(p, c2):
                for d in range(_D // _L):
                    sl = pl.ds(d * _L, _L)
                    w = wpe_v[p, sl]
                    for c in range(_CSEQ):
                        r = c * _SEQ + p
                        rows_v[b, r, sl] = rows_v[b, r, sl] + w
                return c2
            lax.fori_loop(0, _SEQ, body, 0)

        start_remap(0, 0)
        start_remap(1, 1)
        wait_remap(0)
        start_emb(0)

        def step(i, b):
            wait_emb(b)

            @pl.when(i + 1 < _NCHUNK)
            def _():
                @pl.when(i >= 1)
                def _():
                    wait_out(i - 1, 1 - b)
                wait_remap(1 - b)
                start_emb(1 - b)

            @pl.when(i + 2 < _NCHUNK)
            def _():
                start_remap(i + 2, b)

            add_pos(b)
            start_out(i, b)

        def pair(g, carry):
            step(2 * g, 0)
            step(2 * g + 1, 1)
            return carry

        lax.fori_loop(0, _NCHUNK // 2, pair, 0)

        wait_out(_NCHUNK - 2, 0)
        wait_out(_NCHUNK - 1, 1)

    return sc_gather


_SC_CALL = _build_sc_call()


def kernel(input_ids, attention_mask, embedding_dict, input_ids2dict_ids, wpe):
    ids_flat = input_ids.reshape(_BATCH * _SEQ)
    out_flat = _SC_CALL(ids_flat, input_ids2dict_ids, embedding_dict, wpe)
    return out_flat.reshape(_BATCH, _SEQ, _D), attention_mask
